# Initial kernel scaffold; baseline (speedup 1.0000x reference)
import jax
import jax.numpy as jnp
from jax.experimental import pallas as pl


def kernel(hidden_state, attention_mask, ln1_w, ln2_w, wq_a_w, wq_a_b, q_norm_w, wq_b_w, wq_b_b, wkv_a_w, wkv_a_b, kv_norm_w, wkv_b_w, wo_w, wo_b, gate_w, gate_b, eg_w, eu_w, ed_w):
    raise NotImplementedError("write your pallas kernel here")


# speedup vs baseline: 1.1489x; 1.1489x over previous
import math

import jax
import jax.numpy as jnp
from jax.experimental import pallas as pl
from jax.experimental.pallas import tpu as pltpu

_NOPE = 128
_ROPE = 64
_VHD = 128
_EPS = 1e-6


def _rms(x, w):
    var = jnp.mean(x * x, axis=-1, keepdims=True)
    return (x * jax.lax.rsqrt(var + _EPS)) * w


def _rot_half(x):
    half = x.shape[-1] // 2
    return jnp.concatenate([-x[:, half:], x[:, :half]], axis=-1)


def _nt_dot(a, b):
    return jax.lax.dot_general(a, b, (((1,), (1,)), ((), ())),
                               preferred_element_type=jnp.float32)


def _k1_body(nh, nope, rope, hid_ref, cos_ref, sin_ref, ln1_ref, wqa_ref, qnw_ref,
             wqb_ref, wkva_ref, kvnw_ref, wkvbn_ref,
             q2_ref, qpe_ref, kv_ref, kpe_ref):
    x = hid_ref[...]
    xb = _rms(x, ln1_ref[...]).astype(jnp.bfloat16)
    qa = jnp.dot(xb, wqa_ref[...], preferred_element_type=jnp.float32)
    qab = _rms(qa, qnw_ref[...]).astype(jnp.bfloat16)
    q = jnp.dot(qab, wqb_ref[...], preferred_element_type=jnp.float32)
    kvf = jnp.dot(xb, wkva_ref[...], preferred_element_type=jnp.float32)
    kvlr = kvnw_ref.shape[-1]
    kv_ref[...] = _rms(kvf[:, :kvlr], kvnw_ref[...]).astype(jnp.bfloat16)
    cos = cos_ref[...]
    sin = sin_ref[...]
    kpe = kvf[:, kvlr:]
    kpe_ref[...] = (kpe * cos + _rot_half(kpe) * sin).astype(jnp.bfloat16)
    qkhd = nope + rope
    for h in range(nh):
        qn = q[:, h * qkhd:h * qkhd + nope].astype(jnp.bfloat16)
        q2_ref[h] = jnp.dot(qn, wkvbn_ref[h],
                            preferred_element_type=jnp.float32).astype(jnp.bfloat16)
        qp = q[:, h * qkhd + nope:(h + 1) * qkhd]
        qpe_ref[h] = (qp * cos + _rot_half(qp) * sin).astype(jnp.bfloat16)


def _k2_body(scale, q2_ref, qpe_ref, kv_ref, kpe_ref, o_ref):
    qb = pl.program_id(1)
    q2 = q2_ref[0]
    qpe = qpe_ref[0]
    bt = q2.shape[0]
    kvlr = kv_ref.shape[-1]

    def body(j, carry):
        m, l, acc = carry
        kc = kv_ref[pl.ds(j * bt, bt), :]
        pc = kpe_ref[pl.ds(j * bt, bt), :]
        s = _nt_dot(q2, kc) + _nt_dot(qpe, pc)
        s = s * scale
        rows = qb * bt + jax.lax.broadcasted_iota(jnp.int32, (bt, bt), 0)
        cols = j * bt + jax.lax.broadcasted_iota(jnp.int32, (bt, bt), 1)
        s = jnp.where(cols > rows, -1e30, s)
        mnew = jnp.maximum(m, jnp.max(s, axis=-1, keepdims=True))
        p = jnp.exp(s - mnew)
        alpha = jnp.exp(m - mnew)
        lnew = l * alpha + jnp.sum(p, axis=-1, keepdims=True)
        accn = acc * alpha + jnp.dot(p.astype(jnp.bfloat16), kc,
                                     preferred_element_type=jnp.float32)
        return mnew, lnew, accn

    m0 = jnp.full((bt, 1), -1e30, jnp.float32)
    l0 = jnp.zeros((bt, 1), jnp.float32)
    a0 = jnp.zeros((bt, kvlr), jnp.float32)
    m, l, acc = jax.lax.fori_loop(0, qb + 1, body, (m0, l0, a0))
    o_ref[0] = (acc / l).astype(jnp.bfloat16)


def _k3_body(nh, o_ref, wv_ref, woT_ref, wob_ref, hid_ref, ln2_ref, gT_ref, gb_ref,
             h2_ref, y_ref, lg_ref, pr_ref):
    parts = [jnp.dot(o_ref[h], wv_ref[h], preferred_element_type=jnp.float32)
             for h in range(nh)]
    o2 = jnp.concatenate(parts, axis=-1).astype(jnp.bfloat16)
    attn = jnp.dot(o2, woT_ref[...], preferred_element_type=jnp.float32) + wob_ref[...]
    h2 = hid_ref[...] + attn
    h2_ref[...] = h2
    y = _rms(h2, ln2_ref[...])
    yb = y.astype(jnp.bfloat16)
    y_ref[...] = yb
    lg = jnp.dot(yb, gT_ref[...], preferred_element_type=jnp.float32) + gb_ref[...]
    lg_ref[...] = lg
    en = lg.shape[-1]
    col = jax.lax.broadcasted_iota(jnp.int32, lg.shape, 1)
    m1 = jnp.max(lg, axis=-1, keepdims=True)
    i1 = jnp.min(jnp.where(lg == m1, col, en), axis=-1, keepdims=True)
    l2 = jnp.where(col == i1, -jnp.inf, lg)
    m2 = jnp.max(l2, axis=-1, keepdims=True)
    i2 = jnp.min(jnp.where(l2 == m2, col, en), axis=-1, keepdims=True)
    sel = (col == i1) | (col == i2)
    num = jnp.where(sel, jnp.exp(lg - m1), 0.0)
    pr_ref[...] = num / (1.0 + jnp.exp(m2 - m1))


def _k5_body(y_ref, pr_ref, h2_ref, eg_ref, eu_ref, edT_ref, out_ref, acc_ref):
    e = pl.program_id(0)
    tb = pl.program_id(1)
    y = y_ref[...]
    g = jnp.dot(y, eg_ref[0], preferred_element_type=jnp.float32)
    u = jnp.dot(y, eu_ref[0], preferred_element_type=jnp.float32)
    act = (g * jax.nn.sigmoid(g) * u).astype(jnp.bfloat16)
    eo = jnp.dot(act, edT_ref[0], preferred_element_type=jnp.float32)
    col = jax.lax.broadcasted_iota(jnp.int32, pr_ref.shape, 1)
    w = jnp.sum(jnp.where(col == e, pr_ref[...], 0.0), axis=-1, keepdims=True)
    c = eo * w
    bt = y.shape[0]
    sl = pl.ds(tb * bt, bt)

    @pl.when(e == 0)
    def _():
        acc_ref[sl, :] = h2_ref[...] + c

    @pl.when(e > 0)
    def _():
        acc_ref[sl, :] = acc_ref[sl, :] + c

    out_ref[...] = acc_ref[sl, :]


def kernel(hidden_state, attention_mask, ln1_w, ln2_w, wq_a_w, wq_a_b, q_norm_w,
           wq_b_w, wq_b_b, wkv_a_w, wkv_a_b, kv_norm_w, wkv_b_w, wo_w, wo_b,
           gate_w, gate_b, eg_w, eu_w, ed_w):
    bs, S, HS = hidden_state.shape
    hid = hidden_state.reshape(S, HS)
    QLR = wq_a_w.shape[0]
    NH = wq_b_w.shape[0] // (_NOPE + _ROPE)
    KVLR = kv_norm_w.shape[0]
    EN, EK, _ = eg_w.shape
    qkhd = _NOPE + _ROPE
    scale = 1.0 / math.sqrt(float(qkhd))

    inv_freq = 1.0 / (10000.0 ** (jnp.arange(0, _ROPE, 2, dtype=jnp.float32) / _ROPE))
    t = jnp.arange(S, dtype=jnp.float32)[:, None]
    freqs = t * inv_freq[None, :]
    freqs = jnp.concatenate([freqs, freqs], axis=-1)
    cos = jnp.cos(freqs)
    sin = jnp.sin(freqs)

    f16 = jnp.bfloat16
    wqaT = wq_a_w.T.astype(f16)
    wqbT = wq_b_w.T.astype(f16)
    wkvaT = wkv_a_w.T.astype(f16)
    wkvb = wkv_b_w.reshape(NH, _NOPE + _VHD, KVLR)
    wkvbn = wkvb[:, :_NOPE, :].astype(f16)
    wv = wkvb[:, _NOPE:, :].transpose(0, 2, 1).astype(f16)
    woT = wo_w.T.astype(f16)
    gT = gate_w.T.astype(f16)
    egT = eg_w.transpose(0, 2, 1).astype(f16)
    euT = eu_w.transpose(0, 2, 1).astype(f16)
    edT = ed_w.transpose(0, 2, 1).astype(f16)
    ln1 = ln1_w.reshape(1, HS)
    ln2 = ln2_w.reshape(1, HS)
    qnw = q_norm_w.reshape(1, QLR)
    kvnw = kv_norm_w.reshape(1, KVLR)
    wob = wo_b.reshape(1, HS)
    gb = gate_b.reshape(1, EN)

    BT = min(256, S)
    NB = S // BT

    import functools
    k1 = pl.pallas_call(
        functools.partial(_k1_body, NH, _NOPE, _ROPE),
        grid=(NB,),
        in_specs=[
            pl.BlockSpec((BT, HS), lambda i: (i, 0)),
            pl.BlockSpec((BT, _ROPE), lambda i: (i, 0)),
            pl.BlockSpec((BT, _ROPE), lambda i: (i, 0)),
            pl.BlockSpec((1, HS), lambda i: (0, 0)),
            pl.BlockSpec((HS, QLR), lambda i: (0, 0)),
            pl.BlockSpec((1, QLR), lambda i: (0, 0)),
            pl.BlockSpec((QLR, NH * qkhd), lambda i: (0, 0)),
            pl.BlockSpec((HS, KVLR + _ROPE), lambda i: (0, 0)),
            pl.BlockSpec((1, KVLR), lambda i: (0, 0)),
            pl.BlockSpec((NH, _NOPE, KVLR), lambda i: (0, 0, 0)),
        ],
        out_specs=[
            pl.BlockSpec((NH, BT, KVLR), lambda i: (0, i, 0)),
            pl.BlockSpec((NH, BT, _ROPE), lambda i: (0, i, 0)),
            pl.BlockSpec((BT, KVLR), lambda i: (i, 0)),
            pl.BlockSpec((BT, _ROPE), lambda i: (i, 0)),
        ],
        out_shape=[
            jax.ShapeDtypeStruct((NH, S, KVLR), f16),
            jax.ShapeDtypeStruct((NH, S, _ROPE), f16),
            jax.ShapeDtypeStruct((S, KVLR), f16),
            jax.ShapeDtypeStruct((S, _ROPE), f16),
        ],
        compiler_params=pltpu.CompilerParams(
            dimension_semantics=("arbitrary",)),
    )
    q2, qpe, kv, kpe = k1(hid, cos, sin, ln1, wqaT, qnw, wqbT, wkvaT, kvnw, wkvbn)

    k2 = pl.pallas_call(
        functools.partial(_k2_body, scale),
        grid=(NH, NB),
        in_specs=[
            pl.BlockSpec((1, BT, KVLR), lambda h, i: (h, i, 0)),
            pl.BlockSpec((1, BT, _ROPE), lambda h, i: (h, i, 0)),
            pl.BlockSpec((S, KVLR), lambda h, i: (0, 0)),
            pl.BlockSpec((S, _ROPE), lambda h, i: (0, 0)),
        ],
        out_specs=pl.BlockSpec((1, BT, KVLR), lambda h, i: (h, i, 0)),
        out_shape=jax.ShapeDtypeStruct((NH, S, KVLR), f16),
        compiler_params=pltpu.CompilerParams(
            dimension_semantics=("arbitrary", "arbitrary")),
    )
    o = k2(q2, qpe, kv, kpe)

    k3 = pl.pallas_call(
        functools.partial(_k3_body, NH),
        grid=(NB,),
        in_specs=[
            pl.BlockSpec((NH, BT, KVLR), lambda i: (0, i, 0)),
            pl.BlockSpec((NH, KVLR, _VHD), lambda i: (0, 0, 0)),
            pl.BlockSpec((NH * _VHD, HS), lambda i: (0, 0)),
            pl.BlockSpec((1, HS), lambda i: (0, 0)),
            pl.BlockSpec((BT, HS), lambda i: (i, 0)),
            pl.BlockSpec((1, HS), lambda i: (0, 0)),
            pl.BlockSpec((HS, EN), lambda i: (0, 0)),
            pl.BlockSpec((1, EN), lambda i: (0, 0)),
        ],
        out_specs=[
            pl.BlockSpec((BT, HS), lambda i: (i, 0)),
            pl.BlockSpec((BT, HS), lambda i: (i, 0)),
            pl.BlockSpec((BT, EN), lambda i: (i, 0)),
            pl.BlockSpec((BT, EN), lambda i: (i, 0)),
        ],
        out_shape=[
            jax.ShapeDtypeStruct((S, HS), jnp.float32),
            jax.ShapeDtypeStruct((S, HS), f16),
            jax.ShapeDtypeStruct((S, EN), jnp.float32),
            jax.ShapeDtypeStruct((S, EN), jnp.float32),
        ],
        compiler_params=pltpu.CompilerParams(
            dimension_semantics=("arbitrary",)),
    )
    h2, y, logits, probs = k3(o, wv, woT, wob, hid, ln2, gT, gb)

    k5 = pl.pallas_call(
        _k5_body,
        grid=(EN, NB),
        in_specs=[
            pl.BlockSpec((BT, HS), lambda e, i: (i, 0)),
            pl.BlockSpec((BT, EN), lambda e, i: (i, 0)),
            pl.BlockSpec((BT, HS), lambda e, i: (i, 0)),
            pl.BlockSpec((1, HS, EK), lambda e, i: (e, 0, 0)),
            pl.BlockSpec((1, HS, EK), lambda e, i: (e, 0, 0)),
            pl.BlockSpec((1, EK, HS), lambda e, i: (e, 0, 0)),
        ],
        out_specs=pl.BlockSpec((BT, HS), lambda e, i: (i, 0)),
        out_shape=jax.ShapeDtypeStruct((S, HS), jnp.float32),
        scratch_shapes=[pltpu.VMEM((S, HS), jnp.float32)],
        compiler_params=pltpu.CompilerParams(
            dimension_semantics=("arbitrary", "arbitrary")),
    )
    out = k5(y, probs, h2, egT, euT, edT)

    return out.reshape(bs, S, HS), logits
